# baseline (speedup 1.0000x reference)
import functools

import jax
import jax.numpy as jnp
from jax import lax
from jax.experimental import pallas as pl
from jax.experimental.pallas import tpu as pltpu
from jax.experimental.pallas import tpu_sc as plsc

B, K_IN, K_OUT = 2048, 256, 256
IN_DIM, OUT_DIM = 2048, 8192

NC, NS, L = 2, 16, 16
NW = NC * NS
TOK_PER_W = B // NW
CHUNK = 32

_MESH = plsc.VectorSubcoreMesh(core_axis_name="c", subcore_axis_name="s")
_SC_PARAMS = pltpu.CompilerParams(needs_layout_passes=False)

@functools.partial(
    pl.kernel,
    mesh=_MESH,
    out_type=jax.ShapeDtypeStruct((B, IN_DIM), jnp.float32),
    scratch_types=[
        pltpu.VMEM((TOK_PER_W, K_IN), jnp.int32),
        pltpu.VMEM((TOK_PER_W, K_IN), jnp.float32),
        pltpu.VMEM((CHUNK, IN_DIM), jnp.float32),
    ],
    compiler_params=_SC_PARAMS,
)
def _scatter_kernel(vals_hbm, idx_hbm, x_hbm, idx_v, vals_v, rows_v):
    wid = lax.axis_index("s") * NC + lax.axis_index("c")
    tok0 = wid * TOK_PER_W
    pltpu.sync_copy(idx_hbm.at[pl.ds(tok0, TOK_PER_W)], idx_v)
    pltpu.sync_copy(vals_hbm.at[pl.ds(tok0, TOK_PER_W)], vals_v)

    zeros = jnp.zeros((L,), jnp.float32)

    def zero_row(r, c):
        def zero_col(i, c2):
            rows_v[r, pl.ds(i * L, L)] = zeros
            return c2

        lax.fori_loop(0, IN_DIM // L, zero_col, 0)
        return c

    lax.fori_loop(0, CHUNK, zero_row, 0)

    def chunk_body(ci, c):
        def tok_scatter(tl, c2):
            t = ci * CHUNK + tl
            tvec = jnp.full((L,), tl, jnp.int32)
            for g in range(K_IN // L):
                iv = idx_v[t, pl.ds(g * L, L)]
                vv = vals_v[t, pl.ds(g * L, L)]
                plsc.addupdate_scatter(rows_v, [tvec, iv], vv)
            return c2

        lax.fori_loop(0, CHUNK, tok_scatter, 0)
        pltpu.sync_copy(rows_v, x_hbm.at[pl.ds(tok0 + ci * CHUNK, CHUNK)])

        def tok_rezero(tl, c2):
            t = ci * CHUNK + tl
            tvec = jnp.full((L,), tl, jnp.int32)
            for g in range(K_IN // L):
                iv = idx_v[t, pl.ds(g * L, L)]
                plsc.store_scatter(rows_v, [tvec, iv], zeros)
            return c2

        lax.fori_loop(0, CHUNK, tok_rezero, 0)
        return c

    lax.fori_loop(0, TOK_PER_W // CHUNK, chunk_body, 0)


_BM = B
_BN = 1024


def _cast_body(x_ref, o_ref):
    o_ref[...] = x_ref[...].astype(jnp.bfloat16)


def _cast_bf16(x):
    return pl.pallas_call(
        _cast_body,
        grid=(2,),
        in_specs=[pl.BlockSpec((B // 2, IN_DIM), lambda i: (i, 0))],
        out_specs=pl.BlockSpec((B // 2, IN_DIM), lambda i: (i, 0)),
        out_shape=jax.ShapeDtypeStruct((B, IN_DIM), jnp.bfloat16),
    )(x)


def _mm_body(x_ref, w_ref, b_ref, y_ref):
    y_ref[...] = (
        lax.dot_general(
            x_ref[...],
            w_ref[...].astype(jnp.bfloat16),
            dimension_numbers=(((1,), (1,)), ((), ())),
            preferred_element_type=jnp.float32,
        )
        + b_ref[...]
    ).astype(jnp.bfloat16)


def _matmul(x, w, bias2d):
    return pl.pallas_call(
        _mm_body,
        grid=(OUT_DIM // _BN,),
        in_specs=[
            pl.BlockSpec((_BM, IN_DIM), lambda j: (0, 0)),
            pl.BlockSpec((_BN, IN_DIM), lambda j: (j, 0)),
            pl.BlockSpec((1, _BN), lambda j: (0, j)),
        ],
        out_specs=pl.BlockSpec((_BM, _BN), lambda j: (0, j)),
        out_shape=jax.ShapeDtypeStruct((B, OUT_DIM), jnp.bfloat16),
    )(x, w, bias2d)


@functools.partial(
    pl.kernel,
    mesh=_MESH,
    out_type=jax.ShapeDtypeStruct((B, K_OUT), jnp.float32),
    scratch_types=[
        pltpu.VMEM((TOK_PER_W, K_OUT), jnp.int32),
        pltpu.VMEM((TOK_PER_W, K_OUT), jnp.float32),
        pltpu.VMEM((OUT_DIM // 2,), jnp.int32),
        pltpu.VMEM((OUT_DIM // 2,), jnp.int32),
        pltpu.SemaphoreType.DMA,
        pltpu.SemaphoreType.DMA,
    ],
    compiler_params=_SC_PARAMS,
)
def _gather_kernel(y_hbm, idx_hbm, out_hbm, idx_v, out_v, row0, row1, sem0, sem1):
    wid = lax.axis_index("s") * NC + lax.axis_index("c")
    tok0 = wid * TOK_PER_W
    pltpu.sync_copy(idx_hbm.at[pl.ds(tok0, TOK_PER_W)], idx_v)

    rows = (row0, row1)
    sems = (sem0, sem1)
    pltpu.async_copy(y_hbm.at[tok0], row0, sem0)
    pltpu.async_copy(y_hbm.at[tok0 + 1], row1, sem1)

    def pair_body(p, c):
        for bsel in range(2):
            t = p * 2 + bsel
            row, sem = rows[bsel], sems[bsel]
            pltpu.make_async_copy(y_hbm.at[tok0 + t], row, sem).wait()
            for g in range(K_OUT // L):
                o = idx_v[t, pl.ds(g * L, L)]
                w = plsc.load_gather(row, [lax.shift_right_logical(o, 1)])
                even = (o & 1) == 0
                bits = jnp.where(
                    even,
                    lax.shift_left(w, 16),
                    w & jnp.int32(-65536),
                )
                out_v[t, pl.ds(g * L, L)] = plsc.bitcast(bits, jnp.float32)

            @pl.when(t + 2 < TOK_PER_W)
            def _():
                pltpu.async_copy(y_hbm.at[tok0 + t + 2], row, sem)

        return c

    lax.fori_loop(0, TOK_PER_W // 2, pair_body, 0)
    pltpu.sync_copy(out_v, out_hbm.at[pl.ds(tok0, TOK_PER_W)])


def kernel(in_values, active_in_indices, active_out_indices, W, bias):
    in_values = in_values.astype(jnp.float32)
    idx_in = active_in_indices.astype(jnp.int32)
    idx_out = active_out_indices.astype(jnp.int32)
    W = W.astype(jnp.float32)
    bias2d = bias.astype(jnp.float32).reshape(1, OUT_DIM)

    x = _scatter_kernel(in_values, idx_in)
    y = _matmul(_cast_bf16(x), W, bias2d)
    y32 = lax.bitcast_convert_type(
        y.reshape(B, OUT_DIM // 2, 2), jnp.int32
    )
    out = _gather_kernel(y32, idx_out)
    return out


# speedup vs baseline: 2.7233x; 2.7233x over previous
import functools

import jax
import jax.numpy as jnp
from jax import lax
from jax.experimental import pallas as pl
from jax.experimental.pallas import tpu as pltpu
from jax.experimental.pallas import tpu_sc as plsc

B, K_IN, K_OUT = 2048, 256, 256
IN_DIM, OUT_DIM = 2048, 8192

NC, NS, L = 2, 16, 16
NW = NC * NS
TOK_PER_W = B // NW
CHUNK = 16

_MESH = plsc.VectorSubcoreMesh(core_axis_name="c", subcore_axis_name="s")
_SC_PARAMS = pltpu.CompilerParams(needs_layout_passes=False)

@functools.partial(
    pl.kernel,
    mesh=_MESH,
    out_type=jax.ShapeDtypeStruct((B, IN_DIM), jnp.float32),
    scratch_types=[
        pltpu.VMEM((TOK_PER_W, K_IN), jnp.int32),
        pltpu.VMEM((TOK_PER_W, K_IN), jnp.float32),
        pltpu.VMEM((CHUNK, IN_DIM), jnp.float32),
        pltpu.VMEM((CHUNK, IN_DIM), jnp.float32),
        pltpu.SemaphoreType.DMA,
        pltpu.SemaphoreType.DMA,
        pltpu.SemaphoreType.DMA,
    ],
    compiler_params=_SC_PARAMS,
)
def _scatter_kernel(vals_hbm, idx_hbm, x_hbm, idx_v, vals_v, rowsA, rowsB, semI, semA, semB):
    wid = lax.axis_index("s") * NC + lax.axis_index("c")
    tok0 = wid * TOK_PER_W
    pltpu.async_copy(idx_hbm.at[pl.ds(tok0, TOK_PER_W)], idx_v, semI)
    pltpu.async_copy(vals_hbm.at[pl.ds(tok0, TOK_PER_W)], vals_v, semI)

    zeros = jnp.zeros((L,), jnp.float32)
    bufs = (rowsA, rowsB)
    sems = (semA, semB)

    for rows_v in bufs:
        def zero_row(r, c, rows_v=rows_v):
            def zero_col(i, c2):
                rows_v[r, pl.ds(i * L, L)] = zeros
                return c2

            lax.fori_loop(0, IN_DIM // L, zero_col, 0)
            return c

        lax.fori_loop(0, CHUNK, zero_row, 0)

    pltpu.make_async_copy(idx_hbm.at[pl.ds(tok0, TOK_PER_W)], idx_v, semI).wait()
    pltpu.make_async_copy(vals_hbm.at[pl.ds(tok0, TOK_PER_W)], vals_v, semI).wait()

    def scatter_chunk(ci, rows_v):
        def tok_scatter(tl, c2, rows_v=rows_v):
            t = ci * CHUNK + tl
            tvec = jnp.full((L,), tl, jnp.int32)
            for g in range(K_IN // L):
                iv = idx_v[t, pl.ds(g * L, L)]
                vv = vals_v[t, pl.ds(g * L, L)]
                plsc.addupdate_scatter(rows_v, [tvec, iv], vv)
            return c2

        lax.fori_loop(0, CHUNK, tok_scatter, 0)

    def rezero_chunk(ci, rows_v):
        def tok_rezero(tl, c2, rows_v=rows_v):
            t = ci * CHUNK + tl
            tvec = jnp.full((L,), tl, jnp.int32)
            for g in range(K_IN // L):
                iv = idx_v[t, pl.ds(g * L, L)]
                plsc.store_scatter(rows_v, [tvec, iv], zeros)
            return c2

        lax.fori_loop(0, CHUNK, tok_rezero, 0)

    def write_desc(ci, rows_v, sem):
        return pltpu.make_async_copy(
            rows_v, x_hbm.at[pl.ds(tok0 + ci * CHUNK, CHUNK)], sem
        )

    n_chunks = TOK_PER_W // CHUNK
    for ci in range(n_chunks):
        buf, sem = bufs[ci % 2], sems[ci % 2]
        if ci >= 2:
            write_desc(ci - 2, buf, sem).wait()
            rezero_chunk(ci - 2, buf)
        scatter_chunk(ci, buf)
        pltpu.async_copy(buf, x_hbm.at[pl.ds(tok0 + ci * CHUNK, CHUNK)], sem)
    for ci in (n_chunks - 2, n_chunks - 1):
        write_desc(ci, bufs[ci % 2], sems[ci % 2]).wait()


_BM = B
_BN = 1024


def _mm_body(x_ref, w_ref, b_ref, y_ref):
    y_ref[...] = (
        lax.dot_general(
            x_ref[...],
            w_ref[...],
            dimension_numbers=(((1,), (1,)), ((), ())),
            preferred_element_type=jnp.float32,
        )
        + b_ref[...]
    )


def _matmul(x, w, bias2d):
    return pl.pallas_call(
        _mm_body,
        grid=(OUT_DIM // _BN,),
        in_specs=[
            pl.BlockSpec((_BM, IN_DIM), lambda j: (0, 0)),
            pl.BlockSpec((_BN, IN_DIM), lambda j: (j, 0)),
            pl.BlockSpec((1, _BN), lambda j: (0, j)),
        ],
        out_specs=pl.BlockSpec((_BM, _BN), lambda j: (0, j)),
        out_shape=jax.ShapeDtypeStruct((B, OUT_DIM), jnp.float32),
    )(x, w, bias2d)


@functools.partial(
    pl.kernel,
    mesh=_MESH,
    out_type=jax.ShapeDtypeStruct((B, K_OUT), jnp.float32),
    scratch_types=[
        pltpu.VMEM((TOK_PER_W, K_OUT), jnp.int32),
        pltpu.VMEM((TOK_PER_W, K_OUT), jnp.float32),
        pltpu.VMEM((OUT_DIM,), jnp.float32),
        pltpu.VMEM((OUT_DIM,), jnp.float32),
        pltpu.SemaphoreType.DMA,
        pltpu.SemaphoreType.DMA,
    ],
    compiler_params=_SC_PARAMS,
)
def _gather_kernel(y_hbm, idx_hbm, out_hbm, idx_v, out_v, row0, row1, sem0, sem1):
    wid = lax.axis_index("s") * NC + lax.axis_index("c")
    tok0 = wid * TOK_PER_W
    pltpu.sync_copy(idx_hbm.at[pl.ds(tok0, TOK_PER_W)], idx_v)

    rows = (row0, row1)
    sems = (sem0, sem1)
    pltpu.async_copy(y_hbm.at[tok0], row0, sem0)
    pltpu.async_copy(y_hbm.at[tok0 + 1], row1, sem1)

    def pair_body(p, c):
        for bsel in range(2):
            t = p * 2 + bsel
            row, sem = rows[bsel], sems[bsel]
            pltpu.make_async_copy(y_hbm.at[tok0 + t], row, sem).wait()
            for g in range(K_OUT // L):
                o = idx_v[t, pl.ds(g * L, L)]
                out_v[t, pl.ds(g * L, L)] = plsc.load_gather(row, [o])

            @pl.when(t + 2 < TOK_PER_W)
            def _():
                pltpu.async_copy(y_hbm.at[tok0 + t + 2], row, sem)

        return c

    lax.fori_loop(0, TOK_PER_W // 2, pair_body, 0)
    pltpu.sync_copy(out_v, out_hbm.at[pl.ds(tok0, TOK_PER_W)])


def kernel(in_values, active_in_indices, active_out_indices, W, bias):
    in_values = in_values.astype(jnp.float32)
    idx_in = active_in_indices.astype(jnp.int32)
    idx_out = active_out_indices.astype(jnp.int32)
    W = W.astype(jnp.float32)
    bias2d = bias.astype(jnp.float32).reshape(1, OUT_DIM)

    x = _scatter_kernel(in_values, idx_in)
    y = _matmul(x, W, bias2d)
    out = _gather_kernel(y, idx_out)
    return out


# speedup vs baseline: 2.8922x; 1.0620x over previous
import functools

import jax
import jax.numpy as jnp
from jax import lax
from jax.experimental import pallas as pl
from jax.experimental.pallas import tpu as pltpu
from jax.experimental.pallas import tpu_sc as plsc

B, K_IN, K_OUT = 2048, 256, 256
IN_DIM, OUT_DIM = 2048, 8192

NC, NS, L = 2, 16, 16
NW = NC * NS
TOK_PER_W = B // NW
CHUNK = 16

_MESH = plsc.VectorSubcoreMesh(core_axis_name="c", subcore_axis_name="s")
_SC_PARAMS = pltpu.CompilerParams(needs_layout_passes=False)

@functools.partial(
    pl.kernel,
    mesh=_MESH,
    out_type=jax.ShapeDtypeStruct((B, IN_DIM), jnp.float32),
    scratch_types=[
        pltpu.VMEM((TOK_PER_W, K_IN), jnp.int32),
        pltpu.VMEM((TOK_PER_W, K_IN), jnp.float32),
        pltpu.VMEM((CHUNK, IN_DIM), jnp.float32),
        pltpu.VMEM((CHUNK, IN_DIM), jnp.float32),
        pltpu.SemaphoreType.DMA,
        pltpu.SemaphoreType.DMA,
        pltpu.SemaphoreType.DMA,
    ],
    compiler_params=_SC_PARAMS,
)
def _scatter_kernel(vals_hbm, idx_hbm, x_hbm, idx_v, vals_v, rowsA, rowsB, semI, semA, semB):
    wid = lax.axis_index("s") * NC + lax.axis_index("c")
    tok0 = wid * TOK_PER_W
    pltpu.async_copy(idx_hbm.at[pl.ds(tok0, TOK_PER_W)], idx_v, semI)
    pltpu.async_copy(vals_hbm.at[pl.ds(tok0, TOK_PER_W)], vals_v, semI)

    zeros = jnp.zeros((L,), jnp.float32)
    bufs = (rowsA, rowsB)
    sems = (semA, semB)

    for rows_v in bufs:
        def zero_row(r, c, rows_v=rows_v):
            def zero_col(i, c2):
                rows_v[r, pl.ds(i * L, L)] = zeros
                return c2

            lax.fori_loop(0, IN_DIM // L, zero_col, 0)
            return c

        lax.fori_loop(0, CHUNK, zero_row, 0)

    pltpu.make_async_copy(idx_hbm.at[pl.ds(tok0, TOK_PER_W)], idx_v, semI).wait()
    pltpu.make_async_copy(vals_hbm.at[pl.ds(tok0, TOK_PER_W)], vals_v, semI).wait()

    def scatter_chunk(ci, rows_v):
        def tok_scatter(tl, c2, rows_v=rows_v):
            t = ci * CHUNK + tl
            tvec = jnp.full((L,), tl, jnp.int32)
            for g in range(K_IN // L):
                iv = idx_v[t, pl.ds(g * L, L)]
                vv = vals_v[t, pl.ds(g * L, L)]
                plsc.addupdate_scatter(rows_v, [tvec, iv], vv)
            return c2

        lax.fori_loop(0, CHUNK, tok_scatter, 0)

    def rezero_chunk(ci, rows_v):
        def tok_rezero(tl, c2, rows_v=rows_v):
            t = ci * CHUNK + tl
            tvec = jnp.full((L,), tl, jnp.int32)
            for g in range(K_IN // L):
                iv = idx_v[t, pl.ds(g * L, L)]
                plsc.store_scatter(rows_v, [tvec, iv], zeros)
            return c2

        lax.fori_loop(0, CHUNK, tok_rezero, 0)

    def write_desc(ci, rows_v, sem):
        return pltpu.make_async_copy(
            rows_v, x_hbm.at[pl.ds(tok0 + ci * CHUNK, CHUNK)], sem
        )

    n_chunks = TOK_PER_W // CHUNK
    for ci in range(n_chunks):
        buf, sem = bufs[ci % 2], sems[ci % 2]
        if ci >= 2:
            write_desc(ci - 2, buf, sem).wait()
            rezero_chunk(ci - 2, buf)
        scatter_chunk(ci, buf)
        pltpu.async_copy(buf, x_hbm.at[pl.ds(tok0 + ci * CHUNK, CHUNK)], sem)
    for ci in (n_chunks - 2, n_chunks - 1):
        write_desc(ci, bufs[ci % 2], sems[ci % 2]).wait()


_BM = B
_BN = 512
_HALF = OUT_DIM // 2


def _dot(x, w):
    return lax.dot_general(
        x,
        w,
        dimension_numbers=(((1,), (1,)), ((), ())),
        preferred_element_type=jnp.float32,
    )


def _mm_body(x_ref, wlo_ref, whi_ref, blo_ref, bhi_ref, y_ref):
    ylo = _dot(x_ref[...], wlo_ref[...]) + blo_ref[...]
    yhi = _dot(x_ref[...], whi_ref[...]) + bhi_ref[...]
    ulo = lax.bitcast_convert_type(ylo, jnp.int32)
    uhi = lax.bitcast_convert_type(yhi, jnp.int32)
    y_ref[...] = lax.shift_right_logical(ulo, 16) | (uhi & jnp.int32(-65536))


def _matmul(x, w, bias2d):
    nj = _HALF // _BN
    return pl.pallas_call(
        _mm_body,
        grid=(nj,),
        in_specs=[
            pl.BlockSpec((_BM, IN_DIM), lambda j: (0, 0)),
            pl.BlockSpec((_BN, IN_DIM), lambda j: (j, 0)),
            pl.BlockSpec((_BN, IN_DIM), lambda j, nj=nj: (j + nj, 0)),
            pl.BlockSpec((1, _BN), lambda j: (0, j)),
            pl.BlockSpec((1, _BN), lambda j, nj=nj: (0, j + nj)),
        ],
        out_specs=pl.BlockSpec((_BM, _BN), lambda j: (0, j)),
        out_shape=jax.ShapeDtypeStruct((B, _HALF), jnp.int32),
    )(x, w, w, bias2d, bias2d)


@functools.partial(
    pl.kernel,
    mesh=_MESH,
    out_type=jax.ShapeDtypeStruct((B, K_OUT), jnp.float32),
    scratch_types=[
        pltpu.VMEM((TOK_PER_W, K_OUT), jnp.int32),
        pltpu.VMEM((TOK_PER_W, K_OUT), jnp.float32),
        pltpu.VMEM((_HALF,), jnp.int32),
        pltpu.VMEM((_HALF,), jnp.int32),
        pltpu.SemaphoreType.DMA,
        pltpu.SemaphoreType.DMA,
    ],
    compiler_params=_SC_PARAMS,
)
def _gather_kernel(y_hbm, idx_hbm, out_hbm, idx_v, out_v, row0, row1, sem0, sem1):
    wid = lax.axis_index("s") * NC + lax.axis_index("c")
    tok0 = wid * TOK_PER_W
    pltpu.sync_copy(idx_hbm.at[pl.ds(tok0, TOK_PER_W)], idx_v)

    rows = (row0, row1)
    sems = (sem0, sem1)
    pltpu.async_copy(y_hbm.at[tok0], row0, sem0)
    pltpu.async_copy(y_hbm.at[tok0 + 1], row1, sem1)

    def pair_body(p, c):
        for bsel in range(2):
            t = p * 2 + bsel
            row, sem = rows[bsel], sems[bsel]
            pltpu.make_async_copy(y_hbm.at[tok0 + t], row, sem).wait()
            for g in range(K_OUT // L):
                o = idx_v[t, pl.ds(g * L, L)]
                w = plsc.load_gather(row, [o & jnp.int32(_HALF - 1)])
                bits = jnp.where(
                    o < _HALF,
                    lax.shift_left(w, 16),
                    w & jnp.int32(-65536),
                )
                out_v[t, pl.ds(g * L, L)] = plsc.bitcast(bits, jnp.float32)

            @pl.when(t + 2 < TOK_PER_W)
            def _():
                pltpu.async_copy(y_hbm.at[tok0 + t + 2], row, sem)

        return c

    lax.fori_loop(0, TOK_PER_W // 2, pair_body, 0)
    pltpu.sync_copy(out_v, out_hbm.at[pl.ds(tok0, TOK_PER_W)])


def kernel(in_values, active_in_indices, active_out_indices, W, bias):
    in_values = in_values.astype(jnp.float32)
    idx_in = active_in_indices.astype(jnp.int32)
    idx_out = active_out_indices.astype(jnp.int32)
    W = W.astype(jnp.float32)
    bias2d = bias.astype(jnp.float32).reshape(1, OUT_DIM)

    x = _scatter_kernel(in_values, idx_in)
    y = _matmul(x, W, bias2d)
    out = _gather_kernel(y, idx_out)
    return out
